# baseline (speedup 1.0000x reference)
import jax
import jax.numpy as jnp
from jax import lax
from jax.experimental import pallas as pl
from jax.experimental.pallas import tpu as pltpu
from jax.experimental.pallas import tpu_sc as plsc

B = 16384
D = 128
EPS = 1e-6
L = 16

_info = plsc.get_sparse_core_info()
NC = _info.num_cores
NS = _info.num_subcores
NW = NC * NS
BPW = B // NW
CH = 128
NCHUNK = BPW // CH
GROUPS = CH // L
JD = D // L


def _sqrt16(a):
    bits = plsc.bitcast(a, jnp.int32)
    x = plsc.bitcast((bits >> 1) + 0x1FBD1DF5, jnp.float32)
    x = 0.5 * (x + a / x)
    x = 0.5 * (x + a / x)
    return x


def _sc_body(sub_hbm, obj_hbm, idx_hbm, tab_hbm, out_hbm,
             idx_v, s_big, o_big, res_big,
             sem_s, sem_o, sem_g, sem_out):
    wid = lax.axis_index("s") * NC + lax.axis_index("c")
    base = wid * BPW
    lane = lax.iota(jnp.int32, L)

    def s_slice(par):
        return s_big.at[pl.ds(par * CH, CH)]

    def o_slice(par):
        return o_big.at[pl.ds(par * CH, CH)]

    def res_slice(par):
        return res_big.at[pl.ds(par * CH, CH)]

    def sub_src(c):
        return sub_hbm.at[pl.ds(base + c * CH, CH)]

    def obj_src(c):
        return obj_hbm.at[pl.ds(base + c * CH, CH)]

    def gad_src(c):
        return tab_hbm.at[idx_v.at[pl.ds(c * CH, CH)]]

    def start_lin(c, par):
        pltpu.async_copy(sub_src(c), s_slice(par), sem_s)
        pltpu.async_copy(obj_src(c), o_slice(par), sem_o)

    start_lin(0, 0)
    start_lin(1, 1)
    pltpu.sync_copy(idx_hbm.at[pl.ds(base, BPW)], idx_v)
    pltpu.make_async_copy(sub_src(0), s_slice(0), sem_s).wait()
    pltpu.async_copy(gad_src(0), s_slice(0), sem_g, add=True)
    start_lin(2, 2)

    def chunk_body(c, carry):
        par = lax.rem(c, 3)
        npar = lax.rem(c + 1, 3)
        rpar = lax.rem(c, 2)

        @pl.when(c + 1 < NCHUNK)
        def _():
            pltpu.make_async_copy(sub_src(c + 1), s_slice(npar), sem_s).wait()
            pltpu.async_copy(gad_src(c + 1), s_slice(npar), sem_g, add=True)

        pltpu.make_async_copy(gad_src(c), s_slice(par), sem_g).wait()
        pltpu.make_async_copy(obj_src(c), o_slice(par), sem_o).wait()

        @pl.when(c >= 2)
        def _():
            pltpu.make_async_copy(res_slice(rpar),
                                  out_hbm.at[pl.ds(base + (c - 2) * CH, CH)],
                                  sem_out).wait()

        dbase = par * CH
        rbase = rpar * CH

        @plsc.parallel_loop(0, GROUPS, unroll=2)
        def group_body(g):
            tots = [jnp.zeros((L,), jnp.float32) for _ in range(4)]
            for rr in range(L):
                r = dbase + g * L + rr
                acc0 = jnp.zeros((L,), jnp.float32)
                acc1 = jnp.zeros((L,), jnp.float32)
                for j in range(JD):
                    cs = pl.ds(j * L, L)
                    dv = s_big[r, cs] - o_big[r, cs] + EPS
                    if j % 2 == 0:
                        acc0 = acc0 + dv * dv
                    else:
                        acc1 = acc1 + dv * dv
                k = rr % 4
                tots[k] = jnp.where(lane == rr, jnp.sum(acc0 + acc1), tots[k])
            tot = (tots[0] + tots[1]) + (tots[2] + tots[3])
            res_big[pl.ds(rbase + g * L, L)] = _sqrt16(tot)

        pltpu.async_copy(res_slice(rpar),
                         out_hbm.at[pl.ds(base + c * CH, CH)], sem_out)

        @pl.when(c + 3 < NCHUNK)
        def _():
            start_lin(c + 3, par)

        return carry

    lax.fori_loop(0, NCHUNK, chunk_body, 0)

    pltpu.make_async_copy(res_slice(0),
                          out_hbm.at[pl.ds(base + (NCHUNK - 2) * CH, CH)],
                          sem_out).wait()
    pltpu.make_async_copy(res_slice(1),
                          out_hbm.at[pl.ds(base + (NCHUNK - 1) * CH, CH)],
                          sem_out).wait()


_sc_call = pl.kernel(
    _sc_body,
    out_type=jax.ShapeDtypeStruct((B,), jnp.float32),
    mesh=plsc.VectorSubcoreMesh(core_axis_name="c", subcore_axis_name="s"),
    compiler_params=pltpu.CompilerParams(needs_layout_passes=False),
    scratch_types=[
        pltpu.VMEM((BPW,), jnp.int32),
        pltpu.VMEM((3 * CH, D), jnp.float32),
        pltpu.VMEM((3 * CH, D), jnp.float32),
        pltpu.VMEM((2 * CH,), jnp.float32),
        pltpu.SemaphoreType.DMA,
        pltpu.SemaphoreType.DMA,
        pltpu.SemaphoreType.DMA,
        pltpu.SemaphoreType.DMA,
    ],
)


@jax.jit
def kernel(subject_embeddings, object_embeddings, relations, relation_weight):
    rel = relations.astype(jnp.int32)
    return _sc_call(subject_embeddings, object_embeddings, rel,
                    relation_weight)


# speedup vs baseline: 1.0400x; 1.0400x over previous
import jax
import jax.numpy as jnp
from jax import lax
from jax.experimental import pallas as pl
from jax.experimental.pallas import tpu as pltpu
from jax.experimental.pallas import tpu_sc as plsc

B = 16384
D = 128
EPS = 1e-6
L = 16

_info = plsc.get_sparse_core_info()
NC = _info.num_cores
NS = _info.num_subcores
NW = NC * NS
BPW = B // NW
CH = 128
NCHUNK = BPW // CH
GROUPS = CH // L
JD = D // L


def _sqrt16(a):
    bits = plsc.bitcast(a, jnp.int32)
    x = plsc.bitcast((bits >> 1) + 0x1FBD1DF5, jnp.float32)
    x = 0.5 * (x + a / x)
    x = 0.5 * (x + a / x)
    return x


def _sc_body(sub_hbm, obj_hbm, idx_hbm, tab_hbm, out_hbm,
             idx_v, s_big, o_big, res_big,
             sem_s, sem_o, sem_g, sem_out):
    wid = lax.axis_index("s") * NC + lax.axis_index("c")
    base = wid * BPW
    lane = lax.iota(jnp.int32, L)

    def s_slice(par):
        return s_big.at[pl.ds(par * CH, CH)]

    def o_slice(par):
        return o_big.at[pl.ds(par * CH, CH)]

    def res_slice(par):
        return res_big.at[pl.ds(par * CH, CH)]

    def sub_src(c):
        return sub_hbm.at[pl.ds(base + c * CH, CH)]

    def obj_src(c):
        return obj_hbm.at[pl.ds(base + c * CH, CH)]

    def gad_src(c):
        return tab_hbm.at[idx_v.at[pl.ds(c * CH, CH)]]

    def start_lin(c, par):
        pltpu.async_copy(sub_src(c), s_slice(par), sem_s)
        pltpu.async_copy(obj_src(c), o_slice(par), sem_o)

    start_lin(0, 0)
    start_lin(1, 1)
    pltpu.sync_copy(idx_hbm.at[pl.ds(base, BPW)], idx_v)
    pltpu.make_async_copy(sub_src(0), s_slice(0), sem_s).wait()
    pltpu.async_copy(gad_src(0), s_slice(0), sem_g, add=True)
    start_lin(2, 2)

    def chunk_body(c, carry):
        par = lax.rem(c, 3)
        npar = lax.rem(c + 1, 3)

        @pl.when(c + 1 < NCHUNK)
        def _():
            pltpu.make_async_copy(sub_src(c + 1), s_slice(npar), sem_s).wait()
            pltpu.async_copy(gad_src(c + 1), s_slice(npar), sem_g, add=True)

        pltpu.make_async_copy(gad_src(c), s_slice(par), sem_g).wait()
        pltpu.make_async_copy(obj_src(c), o_slice(par), sem_o).wait()

        dbase = par * CH
        rbase = c * CH

        @plsc.parallel_loop(0, GROUPS, unroll=1)
        def group_body(g):
            tots = [jnp.zeros((L,), jnp.float32) for _ in range(4)]
            for rr in range(L):
                r = dbase + g * L + rr
                acc0 = jnp.zeros((L,), jnp.float32)
                acc1 = jnp.zeros((L,), jnp.float32)
                for j in range(JD):
                    cs = pl.ds(j * L, L)
                    dv = s_big[r, cs] - o_big[r, cs] + EPS
                    if j % 2 == 0:
                        acc0 = acc0 + dv * dv
                    else:
                        acc1 = acc1 + dv * dv
                k = rr % 4
                tots[k] = jnp.where(lane == rr, jnp.sum(acc0 + acc1), tots[k])
            tot = (tots[0] + tots[1]) + (tots[2] + tots[3])
            res_big[pl.ds(rbase + g * L, L)] = _sqrt16(tot)

        pltpu.async_copy(res_slice(c),
                         out_hbm.at[pl.ds(base + c * CH, CH)], sem_out)

        @pl.when(c + 3 < NCHUNK)
        def _():
            start_lin(c + 3, par)

        return carry

    lax.fori_loop(0, NCHUNK, chunk_body, 0)

    for c in range(NCHUNK):
        pltpu.make_async_copy(res_slice(c),
                              out_hbm.at[pl.ds(base + c * CH, CH)],
                              sem_out).wait()


_sc_call = pl.kernel(
    _sc_body,
    out_type=jax.ShapeDtypeStruct((B,), jnp.float32),
    mesh=plsc.VectorSubcoreMesh(core_axis_name="c", subcore_axis_name="s"),
    compiler_params=pltpu.CompilerParams(needs_layout_passes=False),
    scratch_types=[
        pltpu.VMEM((BPW,), jnp.int32),
        pltpu.VMEM((3 * CH, D), jnp.float32),
        pltpu.VMEM((3 * CH, D), jnp.float32),
        pltpu.VMEM((BPW,), jnp.float32),
        pltpu.SemaphoreType.DMA,
        pltpu.SemaphoreType.DMA,
        pltpu.SemaphoreType.DMA,
        pltpu.SemaphoreType.DMA,
    ],
)


@jax.jit
def kernel(subject_embeddings, object_embeddings, relations, relation_weight):
    rel = relations.astype(jnp.int32)
    return _sc_call(subject_embeddings, object_embeddings, rel,
                    relation_weight)


# speedup vs baseline: 1.0660x; 1.0250x over previous
import jax
import jax.numpy as jnp
from jax import lax
from jax.experimental import pallas as pl
from jax.experimental.pallas import tpu as pltpu
from jax.experimental.pallas import tpu_sc as plsc

B = 16384
D = 128
EPS = 1e-6
L = 16

_info = plsc.get_sparse_core_info()
NC = _info.num_cores
NS = _info.num_subcores
NW = NC * NS
BPW = B // NW
CH = 64
NCHUNK = BPW // CH
GROUPS = CH // L
JD = D // L


def _sqrt16(a):
    bits = plsc.bitcast(a, jnp.int32)
    x = plsc.bitcast((bits >> 1) + 0x1FBD1DF5, jnp.float32)
    x = 0.5 * (x + a / x)
    x = 0.5 * (x + a / x)
    return x


def _sc_body(sub_hbm, obj_hbm, idx_hbm, tab_hbm, out_hbm,
             idx_v, s_big, o_big, res_big,
             sem_s, sem_o, sem_g, sem_out):
    wid = lax.axis_index("s") * NC + lax.axis_index("c")
    base = wid * BPW
    lane = lax.iota(jnp.int32, L)

    def s_slice(par):
        return s_big.at[pl.ds(par * CH, CH)]

    def o_slice(par):
        return o_big.at[pl.ds(par * CH, CH)]

    def res_slice(par):
        return res_big.at[pl.ds(par * CH, CH)]

    def sub_src(c):
        return sub_hbm.at[pl.ds(base + c * CH, CH)]

    def obj_src(c):
        return obj_hbm.at[pl.ds(base + c * CH, CH)]

    def gad_src(c):
        return tab_hbm.at[idx_v.at[pl.ds(c * CH, CH)]]

    def start_lin(c, par):
        pltpu.async_copy(sub_src(c), s_slice(par), sem_s)
        pltpu.async_copy(obj_src(c), o_slice(par), sem_o)

    start_lin(0, 0)
    start_lin(1, 1)
    pltpu.sync_copy(idx_hbm.at[pl.ds(base, BPW)], idx_v)
    pltpu.make_async_copy(sub_src(0), s_slice(0), sem_s).wait()
    pltpu.async_copy(gad_src(0), s_slice(0), sem_g, add=True)
    start_lin(2, 2)

    def chunk_body(c, carry):
        par = lax.rem(c, 3)
        npar = lax.rem(c + 1, 3)

        @pl.when(c + 1 < NCHUNK)
        def _():
            pltpu.make_async_copy(sub_src(c + 1), s_slice(npar), sem_s).wait()
            pltpu.async_copy(gad_src(c + 1), s_slice(npar), sem_g, add=True)

        pltpu.make_async_copy(gad_src(c), s_slice(par), sem_g).wait()
        pltpu.make_async_copy(obj_src(c), o_slice(par), sem_o).wait()

        dbase = par * CH
        rbase = c * CH

        @plsc.parallel_loop(0, GROUPS, unroll=1)
        def group_body(g):
            tots = [jnp.zeros((L,), jnp.float32) for _ in range(4)]
            for rr in range(L):
                r = dbase + g * L + rr
                acc0 = jnp.zeros((L,), jnp.float32)
                acc1 = jnp.zeros((L,), jnp.float32)
                for j in range(JD):
                    cs = pl.ds(j * L, L)
                    dv = s_big[r, cs] - o_big[r, cs] + EPS
                    if j % 2 == 0:
                        acc0 = acc0 + dv * dv
                    else:
                        acc1 = acc1 + dv * dv
                k = rr % 4
                tots[k] = jnp.where(lane == rr, jnp.sum(acc0 + acc1), tots[k])
            tot = (tots[0] + tots[1]) + (tots[2] + tots[3])
            res_big[pl.ds(rbase + g * L, L)] = _sqrt16(tot)

        pltpu.async_copy(res_slice(c),
                         out_hbm.at[pl.ds(base + c * CH, CH)], sem_out)

        @pl.when(c + 3 < NCHUNK)
        def _():
            start_lin(c + 3, par)

        return carry

    lax.fori_loop(0, NCHUNK, chunk_body, 0)

    for c in range(NCHUNK):
        pltpu.make_async_copy(res_slice(c),
                              out_hbm.at[pl.ds(base + c * CH, CH)],
                              sem_out).wait()


_sc_call = pl.kernel(
    _sc_body,
    out_type=jax.ShapeDtypeStruct((B,), jnp.float32),
    mesh=plsc.VectorSubcoreMesh(core_axis_name="c", subcore_axis_name="s"),
    compiler_params=pltpu.CompilerParams(needs_layout_passes=False),
    scratch_types=[
        pltpu.VMEM((BPW,), jnp.int32),
        pltpu.VMEM((3 * CH, D), jnp.float32),
        pltpu.VMEM((3 * CH, D), jnp.float32),
        pltpu.VMEM((BPW,), jnp.float32),
        pltpu.SemaphoreType.DMA,
        pltpu.SemaphoreType.DMA,
        pltpu.SemaphoreType.DMA,
        pltpu.SemaphoreType.DMA,
    ],
)


@jax.jit
def kernel(subject_embeddings, object_embeddings, relations, relation_weight):
    rel = relations.astype(jnp.int32)
    return _sc_call(subject_embeddings, object_embeddings, rel,
                    relation_weight)
